# baseline (speedup 1.0000x reference)
import jax
import jax.numpy as jnp
from jax import lax
from jax.experimental import pallas as pl
from jax.experimental.pallas import tpu as pltpu
from jax.experimental.pallas import tpu_sc as plsc

NN = 10000
EE = 320000
DD = 128
NC, NS, LANES = 2, 16, 16
NW = NC * NS
CHUNK = 128
GC = 8
EP = 327680
NGR = EP // (GC * CHUNK)
PAIRG = NGR // NS
G0_BIG = 14
G0_SMALL = 8
NP = 10112
RPS = NP // NS
_WB = (128, 128, 128, 128, 120)


def _sc_aggregate(h, srcp, dstp, df, g0):
    mesh = plsc.VectorSubcoreMesh(core_axis_name="c", subcore_axis_name="s")
    g1 = PAIRG - g0
    kb = 8 if df == 16 else 2

    def body(h_ref, src_ref, dst_ref, part_ref, src_v, dst_v, rows_v, acc, sem):
        c = lax.axis_index("c")
        s = lax.axis_index("s")
        zv = jnp.zeros((LANES,), jnp.float32)

        @pl.loop(0, CHUNK)
        def _fill(i):
            for j in range(df // LANES):
                rows_v[i, pl.ds(j * LANES, LANES)] = zv

        base = s * RPS
        off = 0
        for sz in _WB:
            pltpu.sync_copy(rows_v.at[pl.ds(0, sz)], acc.at[pl.ds(base + off, sz)])
            off += sz
        plsc.subcore_barrier()

        cnt = jnp.where(c == 0, g0, g1)
        gbase = jnp.where(c == 0, s * g0, NS * g0 + s * g1)

        @pl.loop(0, cnt)
        def _blk(t):
            pltpu.sync_copy(src_ref.at[gbase + t], src_v)
            pltpu.sync_copy(dst_ref.at[gbase + t], dst_v)

            @pl.loop(0, GC // kb)
            def _chunk(g):
                gd = [pltpu.async_copy(h_ref.at[src_v.at[g * kb + b]],
                                       rows_v.at[pl.ds(b * CHUNK, CHUNK)], sem)
                      for b in range(kb)]
                for d in gd:
                    d.wait()
                sd = [pltpu.async_copy(rows_v.at[pl.ds(b * CHUNK, CHUNK)],
                                       acc.at[dst_v.at[g * kb + b]], sem,
                                       add=True)
                      for b in range(kb)]
                for d in sd:
                    d.wait()

        plsc.subcore_barrier()
        off = 0
        for sz in _WB:
            pltpu.sync_copy(acc.at[pl.ds(base + off, sz)], rows_v.at[pl.ds(0, sz)])
            pltpu.sync_copy(rows_v.at[pl.ds(0, sz)],
                            part_ref.at[c, pl.ds(base + off, sz)])
            off += sz

    fn = pl.kernel(
        body,
        out_type=jax.ShapeDtypeStruct((NC, NP, df), jnp.float32),
        mesh=mesh,
        scratch_types=(
            pltpu.VMEM((GC, CHUNK), jnp.int32),
            pltpu.VMEM((GC, CHUNK), jnp.int32),
            pltpu.VMEM((kb * CHUNK, df), jnp.float32),
            pltpu.VMEM_SHARED((NP, df), jnp.float32),
            pltpu.SemaphoreType.DMA,
        ),
        compiler_params=pltpu.CompilerParams(
            use_tc_tiling_on_sc=(df % 128 == 0)),
    )
    return fn(h, srcp, dstp)


def _gelu(x):
    return 0.5 * x * (1.0 + lax.erf(x * 0.7071067811865476))


def _ln(x, g, b):
    mu = jnp.mean(x, axis=-1, keepdims=True)
    xc = x - mu
    var = jnp.mean(xc * xc, axis=-1, keepdims=True)
    return xc * lax.rsqrt(var + 1e-5) * g + b


_R = 2000


def _tc_layer1(x, p, W, b, g, be):
    def body(h_ref, p_ref, W_ref, b_ref, g_ref, be_ref, o_ref, inv_ref):
        d = p_ref[0, :, DD:DD + 1] + p_ref[1, :, DD:DD + 1]
        inv = 1.0 / jnp.maximum(d, 1.0)
        neigh = (p_ref[0, :, :DD] + p_ref[1, :, :DD]) * inv
        W_ = W_ref[...]
        z = (jnp.dot(h_ref[...], W_[:DD], preferred_element_type=jnp.float32)
             + jnp.dot(neigh, W_[DD:], preferred_element_type=jnp.float32)
             + b_ref[...])
        o_ref[...] = _ln(_gelu(z), g_ref[...], be_ref[...])
        inv_ref[...] = inv

    vec = pl.BlockSpec((1, DD), lambda i: (0, 0))
    return pl.pallas_call(
        body,
        grid=(NN // _R,),
        in_specs=[
            pl.BlockSpec((_R, DD), lambda i: (i, 0)),
            pl.BlockSpec((NC, _R, DD + 16), lambda i: (0, i, 0)),
            pl.BlockSpec((2 * DD, DD), lambda i: (0, 0)),
            vec, vec, vec,
        ],
        out_specs=[pl.BlockSpec((_R, DD), lambda i: (i, 0)),
                   pl.BlockSpec((_R, 1), lambda i: (i, 0))],
        out_shape=[jax.ShapeDtypeStruct((NN, DD), jnp.float32),
                   jax.ShapeDtypeStruct((NN, 1), jnp.float32)],
    )(x, p, W, b.reshape(1, DD), g.reshape(1, DD), be.reshape(1, DD))


def _tc_layer2(h1, p, inv, W, b, g, be, W3, b3):
    def body(h_ref, p_ref, inv_ref, W_ref, b_ref, g_ref, be_ref, w3_ref,
             b3_ref, z3b_ref, a3_ref):
        neigh = (p_ref[0] + p_ref[1]) * inv_ref[...]
        W_ = W_ref[...]
        z = (jnp.dot(h_ref[...], W_[:DD], preferred_element_type=jnp.float32)
             + jnp.dot(neigh, W_[DD:], preferred_element_type=jnp.float32)
             + b_ref[...])
        h2 = _ln(_gelu(z), g_ref[...], be_ref[...])
        w3 = w3_ref[...]
        z3 = jnp.sum(h2 * w3[:, DD:], axis=-1, keepdims=True)
        z3b_ref[...] = jnp.broadcast_to(z3, (_R, 16))
        a3_ref[...] = (jnp.sum(h2 * w3[:, :DD], axis=-1, keepdims=True)
                       + b3_ref[...])

    vec = pl.BlockSpec((1, DD), lambda i: (0, 0))
    return pl.pallas_call(
        body,
        grid=(NN // _R,),
        in_specs=[
            pl.BlockSpec((_R, DD), lambda i: (i, 0)),
            pl.BlockSpec((NC, _R, DD), lambda i: (0, i, 0)),
            pl.BlockSpec((_R, 1), lambda i: (i, 0)),
            pl.BlockSpec((2 * DD, DD), lambda i: (0, 0)),
            vec, vec, vec,
            pl.BlockSpec((1, 2 * DD), lambda i: (0, 0)),
            pl.BlockSpec((1, 1), lambda i: (0, 0)),
        ],
        out_specs=[pl.BlockSpec((_R, 16), lambda i: (i, 0)),
                   pl.BlockSpec((_R, 1), lambda i: (i, 0))],
        out_shape=[jax.ShapeDtypeStruct((NN, 16), jnp.float32),
                   jax.ShapeDtypeStruct((NN, 1), jnp.float32)],
    )(h1, p, inv, W, b.reshape(1, DD), g.reshape(1, DD), be.reshape(1, DD),
      W3.reshape(2 * DD, 1).T, b3.reshape(1, 1))


def _tc_final(a3, p3, inv):
    def body(a3_ref, p3_ref, inv_ref, o_ref):
        sblk = p3_ref[0, :, 0:1] + p3_ref[1, :, 0:1]
        o_ref[...] = a3_ref[...] + sblk * inv_ref[...]

    return pl.pallas_call(
        body,
        grid=(NN // _R,),
        in_specs=[
            pl.BlockSpec((_R, 1), lambda i: (i, 0)),
            pl.BlockSpec((NC, _R, 16), lambda i: (0, i, 0)),
            pl.BlockSpec((_R, 1), lambda i: (i, 0)),
        ],
        out_specs=pl.BlockSpec((_R, 1), lambda i: (i, 0)),
        out_shape=jax.ShapeDtypeStruct((NN, 1), jnp.float32),
    )(a3, p3, inv)


def kernel(x, edge_index, W1, b1, g1, be1, W2, b2, g2, be2, W3, b3):
    src = edge_index[0].astype(jnp.int32)
    dst = edge_index[1].astype(jnp.int32)
    pad = EP - EE
    srcp = jnp.concatenate([src, jnp.zeros((pad,), jnp.int32)]
                           ).reshape(NGR, GC, CHUNK)
    dstp = jnp.concatenate([dst, jnp.full((pad,), NN, jnp.int32)]
                           ).reshape(NGR, GC, CHUNK)

    xt = jnp.concatenate([x, jnp.ones((NN, 16), jnp.float32)], axis=1)
    p1 = _sc_aggregate(xt, srcp, dstp, DD + 16, G0_BIG)
    h1, inv = _tc_layer1(x, p1, W1, b1, g1, be1)
    p2 = _sc_aggregate(h1, srcp, dstp, DD, G0_BIG)
    z3b, a3 = _tc_layer2(h1, p2, inv, W2, b2, g2, be2, W3, b3)
    p3 = _sc_aggregate(z3b, srcp, dstp, 16, G0_SMALL)
    return _tc_final(a3, p3, inv)


# speedup vs baseline: 1.0291x; 1.0291x over previous
import jax
import jax.numpy as jnp
from jax import lax
from jax.experimental import pallas as pl
from jax.experimental.pallas import tpu as pltpu
from jax.experimental.pallas import tpu_sc as plsc

NN = 10000
EE = 320000
DD = 128
NC, NS, LANES = 2, 16, 16
NW = NC * NS
CHUNK = 128
GC = 8
EP = 327680
NGR = EP // (GC * CHUNK)
PAIRG = NGR // NS
G0_BIG = 15
G0_SMALL = 8
NP = 10112
RPS = NP // NS
_WB = (128, 128, 128, 128, 120)


def _sc_aggregate(h, srcp, dstp, df, g0):
    mesh = plsc.VectorSubcoreMesh(core_axis_name="c", subcore_axis_name="s")
    g1 = PAIRG - g0
    kb = 8 if df == 16 else 2

    def body(h_ref, src_ref, dst_ref, part_ref, src_v, dst_v, rows_v, acc, sem):
        c = lax.axis_index("c")
        s = lax.axis_index("s")
        zv = jnp.zeros((LANES,), jnp.float32)

        @pl.loop(0, CHUNK)
        def _fill(i):
            for j in range(df // LANES):
                rows_v[i, pl.ds(j * LANES, LANES)] = zv

        base = s * RPS
        off = 0
        for sz in _WB:
            pltpu.sync_copy(rows_v.at[pl.ds(0, sz)], acc.at[pl.ds(base + off, sz)])
            off += sz
        plsc.subcore_barrier()

        cnt = jnp.where(c == 0, g0, g1)
        gbase = jnp.where(c == 0, s * g0, NS * g0 + s * g1)

        @pl.loop(0, cnt)
        def _blk(t):
            pltpu.sync_copy(src_ref.at[gbase + t], src_v)
            pltpu.sync_copy(dst_ref.at[gbase + t], dst_v)

            @pl.loop(0, GC // kb)
            def _chunk(g):
                gd = [pltpu.async_copy(h_ref.at[src_v.at[g * kb + b]],
                                       rows_v.at[pl.ds(b * CHUNK, CHUNK)], sem)
                      for b in range(kb)]
                for d in gd:
                    d.wait()
                sd = [pltpu.async_copy(rows_v.at[pl.ds(b * CHUNK, CHUNK)],
                                       acc.at[dst_v.at[g * kb + b]], sem,
                                       add=True)
                      for b in range(kb)]
                for d in sd:
                    d.wait()

        plsc.subcore_barrier()
        off = 0
        for sz in _WB:
            pltpu.sync_copy(acc.at[pl.ds(base + off, sz)], rows_v.at[pl.ds(0, sz)])
            pltpu.sync_copy(rows_v.at[pl.ds(0, sz)],
                            part_ref.at[c, pl.ds(base + off, sz)])
            off += sz

    fn = pl.kernel(
        body,
        out_type=jax.ShapeDtypeStruct((NC, NP, df), jnp.float32),
        mesh=mesh,
        scratch_types=(
            pltpu.VMEM((GC, CHUNK), jnp.int32),
            pltpu.VMEM((GC, CHUNK), jnp.int32),
            pltpu.VMEM((kb * CHUNK, df), jnp.float32),
            pltpu.VMEM_SHARED((NP, df), jnp.float32),
            pltpu.SemaphoreType.DMA,
        ),
        compiler_params=pltpu.CompilerParams(
            use_tc_tiling_on_sc=(df % 128 == 0)),
    )
    return fn(h, srcp, dstp)


def _gelu(x):
    return 0.5 * x * (1.0 + lax.erf(x * 0.7071067811865476))


def _ln(x, g, b):
    mu = jnp.mean(x, axis=-1, keepdims=True)
    xc = x - mu
    var = jnp.mean(xc * xc, axis=-1, keepdims=True)
    return xc * lax.rsqrt(var + 1e-5) * g + b


_R = 2000


def _tc_layer1(x, p, W, b, g, be):
    def body(h_ref, p_ref, W_ref, b_ref, g_ref, be_ref, o_ref, inv_ref):
        d = p_ref[0, :, DD:DD + 1] + p_ref[1, :, DD:DD + 1]
        inv = 1.0 / jnp.maximum(d, 1.0)
        neigh = (p_ref[0, :, :DD] + p_ref[1, :, :DD]) * inv
        W_ = W_ref[...]
        z = (jnp.dot(h_ref[...], W_[:DD], preferred_element_type=jnp.float32)
             + jnp.dot(neigh, W_[DD:], preferred_element_type=jnp.float32)
             + b_ref[...])
        o_ref[...] = _ln(_gelu(z), g_ref[...], be_ref[...])
        inv_ref[...] = inv

    vec = pl.BlockSpec((1, DD), lambda i: (0, 0))
    return pl.pallas_call(
        body,
        grid=(NN // _R,),
        in_specs=[
            pl.BlockSpec((_R, DD), lambda i: (i, 0)),
            pl.BlockSpec((NC, _R, DD + 16), lambda i: (0, i, 0)),
            pl.BlockSpec((2 * DD, DD), lambda i: (0, 0)),
            vec, vec, vec,
        ],
        out_specs=[pl.BlockSpec((_R, DD), lambda i: (i, 0)),
                   pl.BlockSpec((_R, 1), lambda i: (i, 0))],
        out_shape=[jax.ShapeDtypeStruct((NN, DD), jnp.float32),
                   jax.ShapeDtypeStruct((NN, 1), jnp.float32)],
    )(x, p, W, b.reshape(1, DD), g.reshape(1, DD), be.reshape(1, DD))


def _tc_layer2(h1, p, inv, W, b, g, be, W3, b3):
    def body(h_ref, p_ref, inv_ref, W_ref, b_ref, g_ref, be_ref, w3_ref,
             b3_ref, z3b_ref, a3_ref):
        neigh = (p_ref[0] + p_ref[1]) * inv_ref[...]
        W_ = W_ref[...]
        z = (jnp.dot(h_ref[...], W_[:DD], preferred_element_type=jnp.float32)
             + jnp.dot(neigh, W_[DD:], preferred_element_type=jnp.float32)
             + b_ref[...])
        h2 = _ln(_gelu(z), g_ref[...], be_ref[...])
        w3 = w3_ref[...]
        z3 = jnp.sum(h2 * w3[:, DD:], axis=-1, keepdims=True)
        z3b_ref[...] = jnp.broadcast_to(z3, (_R, 16))
        a3_ref[...] = (jnp.sum(h2 * w3[:, :DD], axis=-1, keepdims=True)
                       + b3_ref[...])

    vec = pl.BlockSpec((1, DD), lambda i: (0, 0))
    return pl.pallas_call(
        body,
        grid=(NN // _R,),
        in_specs=[
            pl.BlockSpec((_R, DD), lambda i: (i, 0)),
            pl.BlockSpec((NC, _R, DD), lambda i: (0, i, 0)),
            pl.BlockSpec((_R, 1), lambda i: (i, 0)),
            pl.BlockSpec((2 * DD, DD), lambda i: (0, 0)),
            vec, vec, vec,
            pl.BlockSpec((1, 2 * DD), lambda i: (0, 0)),
            pl.BlockSpec((1, 1), lambda i: (0, 0)),
        ],
        out_specs=[pl.BlockSpec((_R, 16), lambda i: (i, 0)),
                   pl.BlockSpec((_R, 1), lambda i: (i, 0))],
        out_shape=[jax.ShapeDtypeStruct((NN, 16), jnp.float32),
                   jax.ShapeDtypeStruct((NN, 1), jnp.float32)],
    )(h1, p, inv, W, b.reshape(1, DD), g.reshape(1, DD), be.reshape(1, DD),
      W3.reshape(2 * DD, 1).T, b3.reshape(1, 1))


def _tc_final(a3, p3, inv):
    def body(a3_ref, p3_ref, inv_ref, o_ref):
        sblk = p3_ref[0, :, 0:1] + p3_ref[1, :, 0:1]
        o_ref[...] = a3_ref[...] + sblk * inv_ref[...]

    return pl.pallas_call(
        body,
        grid=(NN // _R,),
        in_specs=[
            pl.BlockSpec((_R, 1), lambda i: (i, 0)),
            pl.BlockSpec((NC, _R, 16), lambda i: (0, i, 0)),
            pl.BlockSpec((_R, 1), lambda i: (i, 0)),
        ],
        out_specs=pl.BlockSpec((_R, 1), lambda i: (i, 0)),
        out_shape=jax.ShapeDtypeStruct((NN, 1), jnp.float32),
    )(a3, p3, inv)


def kernel(x, edge_index, W1, b1, g1, be1, W2, b2, g2, be2, W3, b3):
    src = edge_index[0].astype(jnp.int32)
    dst = edge_index[1].astype(jnp.int32)
    pad = EP - EE
    srcp = jnp.concatenate([src, jnp.zeros((pad,), jnp.int32)]
                           ).reshape(NGR, GC, CHUNK)
    dstp = jnp.concatenate([dst, jnp.full((pad,), NN, jnp.int32)]
                           ).reshape(NGR, GC, CHUNK)

    xt = jnp.concatenate([x, jnp.ones((NN, 16), jnp.float32)], axis=1)
    p1 = _sc_aggregate(xt, srcp, dstp, DD + 16, G0_BIG)
    h1, inv = _tc_layer1(x, p1, W1, b1, g1, be1)
    p2 = _sc_aggregate(h1, srcp, dstp, DD, G0_BIG)
    z3b, a3 = _tc_layer2(h1, p2, inv, W2, b2, g2, be2, W3, b3)
    p3 = _sc_aggregate(z3b, srcp, dstp, 16, G0_SMALL)
    return _tc_final(a3, p3, inv)


# speedup vs baseline: 1.0394x; 1.0100x over previous
import jax
import jax.numpy as jnp
from jax import lax
from jax.experimental import pallas as pl
from jax.experimental.pallas import tpu as pltpu
from jax.experimental.pallas import tpu_sc as plsc

NN = 10000
EE = 320000
DD = 128
NC, NS, LANES = 2, 16, 16
NW = NC * NS
CHUNK = 128
GC = 8
EP = 327680
NGR = EP // (GC * CHUNK)
PAIRG = NGR // NS
G0_BIG = 15
G0_SMALL = 11
NP = 10112
RPS = NP // NS
_WB = (128, 128, 128, 128, 120)


def _sc_aggregate(h, srcp, dstp, df, g0):
    mesh = plsc.VectorSubcoreMesh(core_axis_name="c", subcore_axis_name="s")
    g1 = PAIRG - g0
    kb = 8 if df == 16 else 2

    def body(h_ref, src_ref, dst_ref, part_ref, src_v, dst_v, rows_v, acc, sem):
        c = lax.axis_index("c")
        s = lax.axis_index("s")
        zv = jnp.zeros((LANES,), jnp.float32)

        @pl.loop(0, CHUNK)
        def _fill(i):
            for j in range(df // LANES):
                rows_v[i, pl.ds(j * LANES, LANES)] = zv

        base = s * RPS
        off = 0
        for sz in _WB:
            pltpu.sync_copy(rows_v.at[pl.ds(0, sz)], acc.at[pl.ds(base + off, sz)])
            off += sz
        plsc.subcore_barrier()

        cnt = jnp.where(c == 0, g0, g1)
        gbase = jnp.where(c == 0, s * g0, NS * g0 + s * g1)

        @pl.loop(0, cnt)
        def _blk(t):
            pltpu.sync_copy(src_ref.at[gbase + t], src_v)
            pltpu.sync_copy(dst_ref.at[gbase + t], dst_v)

            @pl.loop(0, GC // kb)
            def _chunk(g):
                gd = [pltpu.async_copy(h_ref.at[src_v.at[g * kb + b]],
                                       rows_v.at[pl.ds(b * CHUNK, CHUNK)], sem)
                      for b in range(kb)]
                for d in gd:
                    d.wait()
                sd = [pltpu.async_copy(rows_v.at[pl.ds(b * CHUNK, CHUNK)],
                                       acc.at[dst_v.at[g * kb + b]], sem,
                                       add=True)
                      for b in range(kb)]
                for d in sd:
                    d.wait()

        plsc.subcore_barrier()
        off = 0
        for sz in _WB:
            pltpu.sync_copy(acc.at[pl.ds(base + off, sz)], rows_v.at[pl.ds(0, sz)])
            pltpu.sync_copy(rows_v.at[pl.ds(0, sz)],
                            part_ref.at[c, pl.ds(base + off, sz)])
            off += sz

    fn = pl.kernel(
        body,
        out_type=jax.ShapeDtypeStruct((NC, NP, df), jnp.float32),
        mesh=mesh,
        scratch_types=(
            pltpu.VMEM((GC, CHUNK), jnp.int32),
            pltpu.VMEM((GC, CHUNK), jnp.int32),
            pltpu.VMEM((kb * CHUNK, df), jnp.float32),
            pltpu.VMEM_SHARED((NP, df), jnp.float32),
            pltpu.SemaphoreType.DMA,
        ),
        compiler_params=pltpu.CompilerParams(
            use_tc_tiling_on_sc=(df % 128 == 0)),
    )
    return fn(h, srcp, dstp)


def _gelu(x):
    return 0.5 * x * (1.0 + lax.erf(x * 0.7071067811865476))


def _ln(x, g, b):
    mu = jnp.mean(x, axis=-1, keepdims=True)
    xc = x - mu
    var = jnp.mean(xc * xc, axis=-1, keepdims=True)
    return xc * lax.rsqrt(var + 1e-5) * g + b


_R = 2000


def _tc_layer1(x, p, W, b, g, be):
    def body(h_ref, p_ref, W_ref, b_ref, g_ref, be_ref, o_ref, inv_ref):
        d = p_ref[0, :, DD:DD + 1] + p_ref[1, :, DD:DD + 1]
        inv = 1.0 / jnp.maximum(d, 1.0)
        neigh = (p_ref[0, :, :DD] + p_ref[1, :, :DD]) * inv
        W_ = W_ref[...]
        z = (jnp.dot(h_ref[...], W_[:DD], preferred_element_type=jnp.float32)
             + jnp.dot(neigh, W_[DD:], preferred_element_type=jnp.float32)
             + b_ref[...])
        o_ref[...] = _ln(_gelu(z), g_ref[...], be_ref[...])
        inv_ref[...] = inv

    vec = pl.BlockSpec((1, DD), lambda i: (0, 0))
    return pl.pallas_call(
        body,
        grid=(NN // _R,),
        in_specs=[
            pl.BlockSpec((_R, DD), lambda i: (i, 0)),
            pl.BlockSpec((NC, _R, DD + 16), lambda i: (0, i, 0)),
            pl.BlockSpec((2 * DD, DD), lambda i: (0, 0)),
            vec, vec, vec,
        ],
        out_specs=[pl.BlockSpec((_R, DD), lambda i: (i, 0)),
                   pl.BlockSpec((_R, 1), lambda i: (i, 0))],
        out_shape=[jax.ShapeDtypeStruct((NN, DD), jnp.float32),
                   jax.ShapeDtypeStruct((NN, 1), jnp.float32)],
    )(x, p, W, b.reshape(1, DD), g.reshape(1, DD), be.reshape(1, DD))


def _tc_layer2(h1, p, inv, W, b, g, be, W3, b3):
    def body(h_ref, p_ref, inv_ref, W_ref, b_ref, g_ref, be_ref, w3_ref,
             b3_ref, z3b_ref, a3_ref):
        neigh = (p_ref[0] + p_ref[1]) * inv_ref[...]
        W_ = W_ref[...]
        z = (jnp.dot(h_ref[...], W_[:DD], preferred_element_type=jnp.float32)
             + jnp.dot(neigh, W_[DD:], preferred_element_type=jnp.float32)
             + b_ref[...])
        h2 = _ln(_gelu(z), g_ref[...], be_ref[...])
        w3 = w3_ref[...]
        z3 = jnp.sum(h2 * w3[:, DD:], axis=-1, keepdims=True)
        z3b_ref[...] = jnp.broadcast_to(z3, (_R, 16))
        a3_ref[...] = (jnp.sum(h2 * w3[:, :DD], axis=-1, keepdims=True)
                       + b3_ref[...])

    vec = pl.BlockSpec((1, DD), lambda i: (0, 0))
    return pl.pallas_call(
        body,
        grid=(NN // _R,),
        in_specs=[
            pl.BlockSpec((_R, DD), lambda i: (i, 0)),
            pl.BlockSpec((NC, _R, DD), lambda i: (0, i, 0)),
            pl.BlockSpec((_R, 1), lambda i: (i, 0)),
            pl.BlockSpec((2 * DD, DD), lambda i: (0, 0)),
            vec, vec, vec,
            pl.BlockSpec((1, 2 * DD), lambda i: (0, 0)),
            pl.BlockSpec((1, 1), lambda i: (0, 0)),
        ],
        out_specs=[pl.BlockSpec((_R, 16), lambda i: (i, 0)),
                   pl.BlockSpec((_R, 1), lambda i: (i, 0))],
        out_shape=[jax.ShapeDtypeStruct((NN, 16), jnp.float32),
                   jax.ShapeDtypeStruct((NN, 1), jnp.float32)],
    )(h1, p, inv, W, b.reshape(1, DD), g.reshape(1, DD), be.reshape(1, DD),
      W3.reshape(2 * DD, 1).T, b3.reshape(1, 1))


def _tc_final(a3, p3, inv):
    def body(a3_ref, p3_ref, inv_ref, o_ref):
        sblk = p3_ref[0, :, 0:1] + p3_ref[1, :, 0:1]
        o_ref[...] = a3_ref[...] + sblk * inv_ref[...]

    return pl.pallas_call(
        body,
        grid=(NN // _R,),
        in_specs=[
            pl.BlockSpec((_R, 1), lambda i: (i, 0)),
            pl.BlockSpec((NC, _R, 16), lambda i: (0, i, 0)),
            pl.BlockSpec((_R, 1), lambda i: (i, 0)),
        ],
        out_specs=pl.BlockSpec((_R, 1), lambda i: (i, 0)),
        out_shape=jax.ShapeDtypeStruct((NN, 1), jnp.float32),
    )(a3, p3, inv)


def kernel(x, edge_index, W1, b1, g1, be1, W2, b2, g2, be2, W3, b3):
    src = edge_index[0].astype(jnp.int32)
    dst = edge_index[1].astype(jnp.int32)
    pad = EP - EE
    srcp = jnp.concatenate([src, jnp.zeros((pad,), jnp.int32)]
                           ).reshape(NGR, GC, CHUNK)
    dstp = jnp.concatenate([dst, jnp.full((pad,), NN, jnp.int32)]
                           ).reshape(NGR, GC, CHUNK)

    xt = jnp.concatenate([x, jnp.ones((NN, 16), jnp.float32)], axis=1)
    p1 = _sc_aggregate(xt, srcp, dstp, DD + 16, G0_BIG)
    h1, inv = _tc_layer1(x, p1, W1, b1, g1, be1)
    p2 = _sc_aggregate(h1, srcp, dstp, DD, G0_BIG)
    z3b, a3 = _tc_layer2(h1, p2, inv, W2, b2, g2, be2, W3, b3)
    p3 = _sc_aggregate(z3b, srcp, dstp, 16, G0_SMALL)
    return _tc_final(a3, p3, inv)
